# baseline (speedup 1.0000x reference)
import jax
import jax.numpy as jnp
from jax.experimental import pallas as pl
from jax.experimental.pallas import tpu as pltpu

BM = 80


def _fused_body(x_ref, w_ref, adj_ref, b_ref, o_ref, s_ref):
    @pl.when(pl.program_id(0) == 0)
    def _():
        s_ref[...] = jnp.dot(x_ref[...], w_ref[...], preferred_element_type=jnp.float32)

    o_ref[...] = (
        jnp.dot(adj_ref[...], s_ref[...], preferred_element_type=jnp.float32)
        + b_ref[...]
    )


def kernel(input, adj, W, b):
    n, in_dim = input.shape
    out_dim = W.shape[1]

    out = pl.pallas_call(
        _fused_body,
        grid=(n // BM,),
        in_specs=[
            pl.BlockSpec((n, in_dim), lambda i: (0, 0)),
            pl.BlockSpec((in_dim, out_dim), lambda i: (0, 0)),
            pl.BlockSpec((BM, n), lambda i: (i, 0)),
            pl.BlockSpec((1, out_dim), lambda i: (0, 0)),
        ],
        out_specs=pl.BlockSpec((BM, out_dim), lambda i: (i, 0)),
        out_shape=jax.ShapeDtypeStruct((n, out_dim), jnp.float32),
        scratch_shapes=[pltpu.VMEM((n, out_dim), jnp.float32)],
        compiler_params=pltpu.CompilerParams(
            dimension_semantics=("arbitrary",),
        ),
    )(input, W, adj, b.reshape(1, out_dim))
    return out


# speedup vs baseline: 1.3698x; 1.3698x over previous
import jax
import jax.numpy as jnp
from jax.experimental import pallas as pl
from jax.experimental.pallas import tpu as pltpu

BM = 400


def _fused_body(x_ref, w_ref, adj_ref, b_ref, o_ref, s_ref):
    @pl.when(pl.program_id(0) == 0)
    def _():
        s_ref[...] = jnp.dot(
            x_ref[...], w_ref[...], preferred_element_type=jnp.float32
        ).astype(jnp.bfloat16)

    o_ref[...] = (
        jnp.dot(
            adj_ref[...].astype(jnp.bfloat16),
            s_ref[...],
            preferred_element_type=jnp.float32,
        )
        + b_ref[...]
    )


def kernel(input, adj, W, b):
    n, in_dim = input.shape
    out_dim = W.shape[1]

    out = pl.pallas_call(
        _fused_body,
        grid=(n // BM,),
        in_specs=[
            pl.BlockSpec((n, in_dim), lambda i: (0, 0)),
            pl.BlockSpec((in_dim, out_dim), lambda i: (0, 0)),
            pl.BlockSpec((BM, n), lambda i: (i, 0)),
            pl.BlockSpec((1, out_dim), lambda i: (0, 0)),
        ],
        out_specs=pl.BlockSpec((BM, out_dim), lambda i: (i, 0)),
        out_shape=jax.ShapeDtypeStruct((n, out_dim), jnp.float32),
        scratch_shapes=[pltpu.VMEM((n, out_dim), jnp.bfloat16)],
        compiler_params=pltpu.CompilerParams(
            dimension_semantics=("arbitrary",),
        ),
    )(input, W, adj, b.reshape(1, out_dim))
    return out


# speedup vs baseline: 1.3746x; 1.0035x over previous
import jax
import jax.numpy as jnp
from jax.experimental import pallas as pl
from jax.experimental.pallas import tpu as pltpu

BM = 400


def _fused_body(x_ref, w_ref, adj_ref, b_ref, o_ref, s_ref):
    @pl.when(pl.program_id(0) == 0)
    def _():
        s_ref[...] = jnp.dot(x_ref[...], w_ref[...], preferred_element_type=jnp.float32)

    o_ref[...] = (
        jnp.dot(adj_ref[...], s_ref[...], preferred_element_type=jnp.float32)
        + b_ref[...]
    )


def kernel(input, adj, W, b):
    n, in_dim = input.shape
    out_dim = W.shape[1]

    out = pl.pallas_call(
        _fused_body,
        grid=(n // BM,),
        in_specs=[
            pl.BlockSpec((n, in_dim), lambda i: (0, 0)),
            pl.BlockSpec((in_dim, out_dim), lambda i: (0, 0)),
            pl.BlockSpec((BM, n), lambda i: (i, 0)),
            pl.BlockSpec((1, out_dim), lambda i: (0, 0)),
        ],
        out_specs=pl.BlockSpec((BM, out_dim), lambda i: (i, 0)),
        out_shape=jax.ShapeDtypeStruct((n, out_dim), jnp.float32),
        scratch_shapes=[pltpu.VMEM((n, out_dim), jnp.float32)],
        compiler_params=pltpu.CompilerParams(
            dimension_semantics=("arbitrary",),
        ),
    )(input, W, adj, b.reshape(1, out_dim))
    return out
